# baseline (speedup 1.0000x reference)
import functools

import jax
import jax.numpy as jnp
from jax import lax
from jax.experimental import pallas as pl
from jax.experimental.pallas import tpu as pltpu
from jax.experimental.pallas import tpu_sc as plsc
from jax.scipy.linalg import block_diag

_NC = 2
_NS = 16
_NW = _NC * _NS
_LN = 128
_NSC = _NW * _LN
_HR = 64


def _sc_agg(gT, epsvec, half):
    mesh = plsc.VectorSubcoreMesh(core_axis_name="c", subcore_axis_name="s")
    ncols = _NSC // 2

    @functools.partial(
        pl.kernel,
        out_type=jax.ShapeDtypeStruct((_HR, ncols), jnp.float32),
        mesh=mesh,
        scratch_types=[
            pltpu.VMEM((256, _LN), jnp.float32),
            pltpu.VMEM((256, _LN), jnp.float32),
            pltpu.VMEM((256, _LN), jnp.float32),
            pltpu.VMEM((_HR, _LN), jnp.float32),
            pltpu.VMEM((16,), jnp.float32),
            pltpu.SemaphoreType.DMA,
            pltpu.SemaphoreType.DMA,
            pltpu.SemaphoreType.DMA,
        ],
    )
    def agg(g_hbm, eps_hbm, h_hbm, qbuf0, qbuf1, qbuf2, obuf, epsv,
            sem0, sem1, sem2):
        wid = lax.axis_index("s") * _NC + lax.axis_index("c")
        active = (wid // 16) == half
        col0 = wid * _LN
        lcol0 = col0 - half * ncols

        @pl.when(active)
        def _():
            pltpu.sync_copy(eps_hbm, epsv)
            ev = epsv[...]
            bufs = (qbuf0, qbuf1, qbuf2)
            sems = (sem0, sem1, sem2)

            def src(q):
                return g_hbm.at[pl.ds(q * 256, 256), pl.ds(col0, _LN)]

            handles = [pltpu.async_copy(src(0), bufs[0], sems[0]),
                       pltpu.async_copy(src(1), bufs[1], sems[1])]
            for q in range(4):
                handles[q].wait()
                if q + 2 < 4:
                    handles.append(
                        pltpu.async_copy(src(q + 2), bufs[(q + 2) % 3],
                                         sems[(q + 2) % 3]))
                b = bufs[q % 3]

                if q == 0:
                    def fbody(f, carry):
                        for l in range(8):
                            sl = pl.ds(l * 16, 16)
                            acc = b[f, sl] * ev
                            acc = acc + b[64 + f, sl]
                            acc = acc + b[128 + f, sl]
                            acc = acc + b[192 + f, sl]
                            obuf[f, sl] = acc
                        return carry
                else:
                    def fbody(f, carry):
                        for l in range(8):
                            sl = pl.ds(l * 16, 16)
                            acc = b[f, sl] + b[64 + f, sl]
                            acc = acc + b[128 + f, sl]
                            acc = acc + b[192 + f, sl]
                            obuf[f, sl] = obuf[f, sl] + acc
                        return carry

                lax.fori_loop(0, _HR, fbody, 0)

            pltpu.sync_copy(obuf, h_hbm.at[:, pl.ds(lcol0, _LN)])

    return agg(gT, epsvec)


def _fused_body(g_ref, AT_ref, W1_ref, b1_ref, W2_ref, b2_ref,
                rW1_ref, rb1_ref, rW2_ref, rb2_ref, o_ref):
    hT = jnp.dot(AT_ref[...], g_ref[...], preferred_element_type=jnp.float32)
    aT = jnp.dot(W1_ref[...], hT, preferred_element_type=jnp.float32)
    b1v = b1_ref[...]
    uT = jnp.maximum(aT + b1v, 0.0) + jnp.maximum(b1v - aT, 0.0)
    zT = jnp.dot(W2_ref[...], uT, preferred_element_type=jnp.float32) + b2_ref[...]
    tT = jnp.maximum(
        jnp.dot(rW1_ref[...], zT, preferred_element_type=jnp.float32) + rb1_ref[...],
        0.0)
    o_ref[...] = jnp.dot(rW2_ref[...], tT, preferred_element_type=jnp.float32) + rb2_ref[...]


def _mlp_body(h_ref, W1_ref, b1_ref, W2_ref, b2_ref,
              rW1_ref, rb1_ref, rW2_ref, rb2_ref, o_ref):
    aT = jnp.dot(W1_ref[...], h_ref[...], preferred_element_type=jnp.float32)
    b1v = b1_ref[...]
    uT = jnp.maximum(aT + b1v, 0.0) + jnp.maximum(b1v - aT, 0.0)
    zT = jnp.dot(W2_ref[...], uT, preferred_element_type=jnp.float32) + b2_ref[...]
    tT = jnp.maximum(
        jnp.dot(rW1_ref[...], zT, preferred_element_type=jnp.float32) + rb1_ref[...],
        0.0)
    o_ref[...] = jnp.dot(rW2_ref[...], tT, preferred_element_type=jnp.float32) + rb2_ref[...]


_WSPECS = [
    pl.BlockSpec((256, 64), lambda i: (0, 0)),
    pl.BlockSpec((256, 1), lambda i: (0, 0)),
    pl.BlockSpec((128, 256), lambda i: (0, 0)),
    pl.BlockSpec((128, 1), lambda i: (0, 0)),
    pl.BlockSpec((64, 128), lambda i: (0, 0)),
    pl.BlockSpec((64, 1), lambda i: (0, 0)),
    pl.BlockSpec((32, 64), lambda i: (0, 0)),
    pl.BlockSpec((32, 1), lambda i: (0, 0)),
]


def kernel(g, x, eps, enc_W1, enc_b1, enc_W2, enc_b2,
           rho_W1, rho_b1, rho_W2, rho_b2):
    B, N, S, M, D = g.shape
    H = enc_W1.shape[1]
    O = enc_W2.shape[1]
    MD = M * D
    NB = B * N
    SMD = S * MD

    gT = jnp.transpose(g, (0, 2, 3, 4, 1)).reshape(SMD, NB)
    epsvec = jnp.full((16,), 2.0 + eps, jnp.float32)

    hT_sc1 = _sc_agg(gT, epsvec, 0)
    hT_sc2 = _sc_agg(gT, epsvec, 1)

    coef = jnp.ones((S,), g.dtype).at[0].add(1.0 + eps)
    A = (coef[:, None, None] * jnp.eye(MD, dtype=g.dtype)).reshape(SMD, MD)
    AT = A.T
    W1bigT = block_diag(*([enc_W1.T] * M))
    b1bigT = jnp.tile(enc_b1, M)[:, None]
    W2bigT = block_diag(*([enc_W2.T] * M))
    b2bigT = jnp.tile(2.0 * enc_b2, M)[:, None]
    weights = (W1bigT, b1bigT, W2bigT, b2bigT,
               rho_W1.T, rho_b1[:, None], rho_W2.T, rho_b2[:, None])

    BN_A = 2048
    n_tc = NB - _NSC
    base_blk = _NSC // BN_A
    grid_a = -(-n_tc // BN_A)
    out_tc = pl.pallas_call(
        _fused_body,
        grid=(grid_a,),
        in_specs=[
            pl.BlockSpec((SMD, BN_A), lambda i: (0, i + base_blk)),
            pl.BlockSpec((MD, SMD), lambda i: (0, 0)),
        ] + _WSPECS,
        out_specs=pl.BlockSpec((O, BN_A), lambda i: (0, i)),
        out_shape=jax.ShapeDtypeStruct((O, n_tc), g.dtype),
    )(gT, AT, *weights)

    BN_B = _NSC // 2
    mlp_b = functools.partial(
        pl.pallas_call,
        _mlp_body,
        grid=(1,),
        in_specs=[pl.BlockSpec((MD, BN_B), lambda i: (0, 0))] + _WSPECS,
        out_specs=pl.BlockSpec((O, BN_B), lambda i: (0, 0)),
        out_shape=jax.ShapeDtypeStruct((O, BN_B), g.dtype),
    )
    out_sc1 = mlp_b()(hT_sc1, *weights)
    out_sc2 = mlp_b()(hT_sc2, *weights)

    outT = jnp.concatenate([out_sc1, out_sc2, out_tc], axis=1)
    return outT.T.reshape(B, N, O)


# speedup vs baseline: 1.2092x; 1.2092x over previous
import functools

import jax
import jax.numpy as jnp
from jax import lax
from jax.experimental import pallas as pl
from jax.experimental.pallas import tpu as pltpu
from jax.experimental.pallas import tpu_sc as plsc
from jax.scipy.linalg import block_diag

_NC = 2
_NS = 16
_NW = _NC * _NS
_LN = 128
_NSC = _NW * _LN
_HR = 64


def _sc_agg(gT, epsvec):
    mesh = plsc.VectorSubcoreMesh(core_axis_name="c", subcore_axis_name="s")

    @functools.partial(
        pl.kernel,
        out_type=jax.ShapeDtypeStruct((_HR, _NSC), jnp.float32),
        mesh=mesh,
        scratch_types=[
            pltpu.VMEM((256, _LN), jnp.float32),
            pltpu.VMEM((256, _LN), jnp.float32),
            pltpu.VMEM((256, _LN), jnp.float32),
            pltpu.VMEM((_HR, _LN), jnp.float32),
            pltpu.VMEM((16,), jnp.float32),
            pltpu.SemaphoreType.DMA,
            pltpu.SemaphoreType.DMA,
            pltpu.SemaphoreType.DMA,
        ],
    )
    def agg(g_hbm, eps_hbm, h_hbm, qbuf0, qbuf1, qbuf2, obuf, epsv,
            sem0, sem1, sem2):
        wid = lax.axis_index("s") * _NC + lax.axis_index("c")
        col0 = wid * _LN
        pltpu.sync_copy(eps_hbm, epsv)
        ev = epsv[...]
        bufs = (qbuf0, qbuf1, qbuf2)
        sems = (sem0, sem1, sem2)

        def src(q):
            return g_hbm.at[pl.ds(q * 256, 256), pl.ds(col0, _LN)]

        handles = [pltpu.async_copy(src(0), bufs[0], sems[0]),
                   pltpu.async_copy(src(1), bufs[1], sems[1])]
        for q in range(4):
            handles[q].wait()
            if q + 2 < 4:
                handles.append(
                    pltpu.async_copy(src(q + 2), bufs[(q + 2) % 3],
                                     sems[(q + 2) % 3]))
            b = bufs[q % 3]

            if q == 0:
                def fbody(f, carry):
                    for l in range(8):
                        sl = pl.ds(l * 16, 16)
                        acc = b[f, sl] * ev
                        acc = acc + b[64 + f, sl]
                        acc = acc + b[128 + f, sl]
                        acc = acc + b[192 + f, sl]
                        obuf[f, sl] = acc
                    return carry
            else:
                def fbody(f, carry):
                    for l in range(8):
                        sl = pl.ds(l * 16, 16)
                        acc = b[f, sl] + b[64 + f, sl]
                        acc = acc + b[128 + f, sl]
                        acc = acc + b[192 + f, sl]
                        obuf[f, sl] = obuf[f, sl] + acc
                    return carry

            lax.fori_loop(0, _HR, fbody, 0)

        pltpu.sync_copy(obuf, h_hbm.at[:, pl.ds(col0, _LN)])

    return agg(gT, epsvec)


def _fused_body(g_ref, AT_ref, W1_ref, b1_ref, W2_ref, b2_ref,
                rW1_ref, rb1_ref, rW2_ref, rb2_ref, o_ref):
    hT = jnp.dot(AT_ref[...], g_ref[...], preferred_element_type=jnp.float32)
    aT = jnp.dot(W1_ref[...], hT, preferred_element_type=jnp.float32)
    b1v = b1_ref[...]
    uT = jnp.maximum(aT + b1v, 0.0) + jnp.maximum(b1v - aT, 0.0)
    zT = jnp.dot(W2_ref[...], uT, preferred_element_type=jnp.float32) + b2_ref[...]
    tT = jnp.maximum(
        jnp.dot(rW1_ref[...], zT, preferred_element_type=jnp.float32) + rb1_ref[...],
        0.0)
    o_ref[...] = jnp.dot(rW2_ref[...], tT, preferred_element_type=jnp.float32) + rb2_ref[...]


def _mlp_body(h_ref, W1_ref, b1_ref, W2_ref, b2_ref,
              rW1_ref, rb1_ref, rW2_ref, rb2_ref, o_ref):
    aT = jnp.dot(W1_ref[...], h_ref[...], preferred_element_type=jnp.float32)
    b1v = b1_ref[...]
    uT = jnp.maximum(aT + b1v, 0.0) + jnp.maximum(b1v - aT, 0.0)
    zT = jnp.dot(W2_ref[...], uT, preferred_element_type=jnp.float32) + b2_ref[...]
    tT = jnp.maximum(
        jnp.dot(rW1_ref[...], zT, preferred_element_type=jnp.float32) + rb1_ref[...],
        0.0)
    o_ref[...] = jnp.dot(rW2_ref[...], tT, preferred_element_type=jnp.float32) + rb2_ref[...]


_WSPECS = [
    pl.BlockSpec((256, 64), lambda i: (0, 0)),
    pl.BlockSpec((256, 1), lambda i: (0, 0)),
    pl.BlockSpec((128, 256), lambda i: (0, 0)),
    pl.BlockSpec((128, 1), lambda i: (0, 0)),
    pl.BlockSpec((64, 128), lambda i: (0, 0)),
    pl.BlockSpec((64, 1), lambda i: (0, 0)),
    pl.BlockSpec((32, 64), lambda i: (0, 0)),
    pl.BlockSpec((32, 1), lambda i: (0, 0)),
]


def kernel(g, x, eps, enc_W1, enc_b1, enc_W2, enc_b2,
           rho_W1, rho_b1, rho_W2, rho_b2):
    B, N, S, M, D = g.shape
    H = enc_W1.shape[1]
    O = enc_W2.shape[1]
    MD = M * D
    NB = B * N
    SMD = S * MD

    gT = jnp.transpose(g, (0, 2, 3, 4, 1)).reshape(SMD, NB)
    epsvec = jnp.full((16,), 2.0 + eps, jnp.float32)

    hT_sc = _sc_agg(gT, epsvec)

    coef = jnp.ones((S,), g.dtype).at[0].add(1.0 + eps)
    A = (coef[:, None, None] * jnp.eye(MD, dtype=g.dtype)).reshape(SMD, MD)
    AT = A.T
    W1bigT = block_diag(*([enc_W1.T] * M))
    b1bigT = jnp.tile(enc_b1, M)[:, None]
    W2bigT = block_diag(*([enc_W2.T] * M))
    b2bigT = jnp.tile(2.0 * enc_b2, M)[:, None]
    weights = (W1bigT, b1bigT, W2bigT, b2bigT,
               rho_W1.T, rho_b1[:, None], rho_W2.T, rho_b2[:, None])

    BN_A = 2048
    n_tc = NB - _NSC
    base_blk = _NSC // BN_A
    grid_a = -(-n_tc // BN_A)
    out_tc = pl.pallas_call(
        _fused_body,
        grid=(grid_a,),
        in_specs=[
            pl.BlockSpec((SMD, BN_A), lambda i: (0, i + base_blk)),
            pl.BlockSpec((MD, SMD), lambda i: (0, 0)),
        ] + _WSPECS,
        out_specs=pl.BlockSpec((O, BN_A), lambda i: (0, i)),
        out_shape=jax.ShapeDtypeStruct((O, n_tc), g.dtype),
    )(gT, AT, *weights)

    out_sc = pl.pallas_call(
        _mlp_body,
        grid=(1,),
        in_specs=[pl.BlockSpec((MD, _NSC), lambda i: (0, 0))] + _WSPECS,
        out_specs=pl.BlockSpec((O, _NSC), lambda i: (0, 0)),
        out_shape=jax.ShapeDtypeStruct((O, _NSC), g.dtype),
    )(hT_sc, *weights)

    outT = jnp.concatenate([out_sc, out_tc], axis=1)
    return outT.T.reshape(B, N, O)
